# Initial kernel scaffold; baseline (speedup 1.0000x reference)
import jax
import jax.numpy as jnp
from jax.experimental import pallas as pl


def kernel(input_ids, token_weight, pos_weight):
    raise NotImplementedError("write your pallas kernel here")


# speedup vs baseline: 2.2381x; 2.2381x over previous
import functools

import jax
import jax.numpy as jnp
from jax import lax
from jax.experimental import pallas as pl
from jax.experimental.pallas import tpu as pltpu
from jax.experimental.pallas import tpu_sc as plsc

_LANES = 16
_CHUNK = 128


def _build(B, S, V, D, NC, NS):
    N = B * S
    NW = NC * NS
    rows_per_w = N // NW
    n_chunks = rows_per_w // _CHUNK
    mesh = plsc.VectorSubcoreMesh(core_axis_name="c", subcore_axis_name="s")

    @functools.partial(
        pl.kernel,
        mesh=mesh,
        out_type=jax.ShapeDtypeStruct((N, D), jnp.float32),
        scratch_types=[
            pltpu.VMEM((n_chunks, _CHUNK), jnp.int32),
            pltpu.VMEM((_CHUNK, D), jnp.float32),
            pltpu.VMEM((_CHUNK, D), jnp.float32),
            pltpu.SemaphoreType.DMA,
            pltpu.SemaphoreType.DMA,
        ],
    )
    def emb(ids_hbm, tok_hbm, pos_hbm, out_hbm, idx_v, tok_v, pos_v, sem_t, sem_p):
        wid = lax.axis_index("s") * NC + lax.axis_index("c")
        base = wid * rows_per_w
        pos_base = lax.rem(base, S)
        pltpu.sync_copy(ids_hbm.at[pl.ds(wid * n_chunks, n_chunks)], idx_v)
        for j in range(n_chunks):
            cp_t = pltpu.async_copy(tok_hbm.at[idx_v.at[j]], tok_v, sem_t)
            cp_p = pltpu.async_copy(
                pos_hbm.at[pl.ds(pos_base + j * _CHUNK, _CHUNK)], pos_v, sem_p
            )
            cp_t.wait()
            cp_p.wait()

            def add_row(r, carry):
                for c in range(D // _LANES):
                    v = pos_v[r, pl.ds(c * _LANES, _LANES)]
                    plsc.addupdate(tok_v.at[r, pl.ds(c * _LANES, _LANES)], v)
                return carry

            lax.fori_loop(0, _CHUNK, add_row, 0)
            pltpu.sync_copy(tok_v, out_hbm.at[pl.ds(base + j * _CHUNK, _CHUNK)])

    return emb


def kernel(input_ids, token_weight, pos_weight):
    B, S = input_ids.shape
    V, D = token_weight.shape
    info = plsc.get_sparse_core_info()
    NC, NS = info.num_cores, info.num_subcores
    n_chunks = (B * S) // (NC * NS * _CHUNK)
    ids2d = input_ids.astype(jnp.int32).reshape(NC * NS * n_chunks, _CHUNK)
    emb = _build(B, S, V, D, NC, NS)
    out = emb(ids2d, token_weight, pos_weight)
    return out.reshape(B, S, D)


# speedup vs baseline: 2.7062x; 1.2091x over previous
import functools

import jax
import jax.numpy as jnp
from jax import lax
from jax.experimental import pallas as pl
from jax.experimental.pallas import tpu as pltpu
from jax.experimental.pallas import tpu_sc as plsc

_LANES = 16
_CHUNK = 128


def _build(B, S, V, D, NC, NS):
    N = B * S
    mesh = plsc.VectorSubcoreMesh(core_axis_name="c", subcore_axis_name="s")

    @functools.partial(
        pl.kernel,
        mesh=mesh,
        out_type=jax.ShapeDtypeStruct((N, D), jnp.float32),
        scratch_types=[
            pltpu.VMEM((B, _CHUNK), jnp.int32),
            pltpu.VMEM((_CHUNK, D), jnp.float32),
            pltpu.VMEM((2, _CHUNK, D), jnp.float32),
            pltpu.SemaphoreType.DMA,
            pltpu.SemaphoreType.DMA,
            pltpu.SemaphoreType.DMA,
        ],
    )
    def emb(ids_hbm, tok_hbm, pos_hbm, out_hbm, idx_v, pos_v, tok_v, sem_p, sem_g, sem_s):
        wid = lax.axis_index("s") * NC + lax.axis_index("c")
        off = wid * _CHUNK
        pos_cp = pltpu.async_copy(pos_hbm.at[pl.ds(off, _CHUNK)], pos_v, sem_p)
        for b in range(B):
            pltpu.sync_copy(ids_hbm.at[b, pl.ds(off, _CHUNK)], idx_v.at[b])
        gathers = {0: pltpu.async_copy(tok_hbm.at[idx_v.at[0]], tok_v.at[0], sem_g)}
        pos_cp.wait()
        stores = {}
        for b in range(B):
            cur = b % 2
            gathers[b].wait()
            if b + 1 < B:
                if b - 1 >= 0:
                    stores[b - 1].wait()
                gathers[b + 1] = pltpu.async_copy(
                    tok_hbm.at[idx_v.at[b + 1]], tok_v.at[(b + 1) % 2], sem_g
                )

            def add_row(r, carry):
                for c in range(D // _LANES):
                    v = pos_v[r, pl.ds(c * _LANES, _LANES)]
                    plsc.addupdate(tok_v.at[cur, r, pl.ds(c * _LANES, _LANES)], v)
                return carry

            lax.fori_loop(0, _CHUNK, add_row, 0)
            stores[b] = pltpu.async_copy(
                tok_v.at[cur], out_hbm.at[pl.ds(b * S + off, _CHUNK)], sem_s
            )
        stores[B - 2].wait()
        stores[B - 1].wait()

    return emb


def kernel(input_ids, token_weight, pos_weight):
    B, S = input_ids.shape
    V, D = token_weight.shape
    info = plsc.get_sparse_core_info()
    NC, NS = info.num_cores, info.num_subcores
    emb = _build(B, S, V, D, NC, NS)
    out = emb(input_ids.astype(jnp.int32), token_weight, pos_weight)
    return out.reshape(B, S, D)


# speedup vs baseline: 2.7398x; 1.0124x over previous
import functools

import jax
import jax.numpy as jnp
from jax import lax
from jax.experimental import pallas as pl
from jax.experimental.pallas import tpu as pltpu
from jax.experimental.pallas import tpu_sc as plsc

_LANES = 16
_CHUNK = 128


def _build(B, S, V, D, NC, NS):
    N = B * S
    mesh = plsc.VectorSubcoreMesh(core_axis_name="c", subcore_axis_name="s")

    @functools.partial(
        pl.kernel,
        mesh=mesh,
        out_type=jax.ShapeDtypeStruct((N, D), jnp.float32),
        scratch_types=[
            pltpu.VMEM((B, _CHUNK), jnp.int32),
            pltpu.VMEM((_CHUNK, D), jnp.float32),
            pltpu.VMEM((2, _CHUNK, D), jnp.float32),
            pltpu.SemaphoreType.DMA,
            pltpu.SemaphoreType.DMA,
            pltpu.SemaphoreType.DMA,
            pltpu.SemaphoreType.DMA,
        ],
    )
    def emb(ids_hbm, tok_hbm, pos_hbm, out_hbm, idx_v, pos_v, tok_v, sem_p, sem_i, sem_g, sem_s):
        wid = lax.axis_index("s") * NC + lax.axis_index("c")
        off = wid * _CHUNK
        pos_cp = pltpu.async_copy(pos_hbm.at[pl.ds(off, _CHUNK)], pos_v, sem_p)
        idx_cps = [
            pltpu.async_copy(ids_hbm.at[b, pl.ds(off, _CHUNK)], idx_v.at[b], sem_i)
            for b in range(B)
        ]
        idx_cps[0].wait()
        gathers = {0: pltpu.async_copy(tok_hbm.at[idx_v.at[0]], tok_v.at[0], sem_g)}
        for b in range(1, B):
            idx_cps[b].wait()
        pos_cp.wait()
        stores = {}
        for b in range(B):
            cur = b % 2
            gathers[b].wait()
            if b + 1 < B:
                if b - 1 >= 0:
                    stores[b - 1].wait()
                gathers[b + 1] = pltpu.async_copy(
                    tok_hbm.at[idx_v.at[b + 1]], tok_v.at[(b + 1) % 2], sem_g
                )

            @plsc.parallel_loop(0, _CHUNK, step=1, unroll=4)
            def add_row(r):
                for c in range(D // _LANES):
                    v = pos_v[r, pl.ds(c * _LANES, _LANES)]
                    plsc.addupdate(tok_v.at[cur, r, pl.ds(c * _LANES, _LANES)], v)

            stores[b] = pltpu.async_copy(
                tok_v.at[cur], out_hbm.at[pl.ds(b * S + off, _CHUNK)], sem_s
            )
        stores[B - 2].wait()
        stores[B - 1].wait()

    return emb


def kernel(input_ids, token_weight, pos_weight):
    B, S = input_ids.shape
    V, D = token_weight.shape
    info = plsc.get_sparse_core_info()
    NC, NS = info.num_cores, info.num_subcores
    emb = _build(B, S, V, D, NC, NS)
    out = emb(input_ids.astype(jnp.int32), token_weight, pos_weight)
    return out.reshape(B, S, D)


# speedup vs baseline: 2.9134x; 1.0634x over previous
import functools

import jax
import jax.numpy as jnp
from jax import lax
from jax.experimental import pallas as pl
from jax.experimental.pallas import tpu as pltpu
from jax.experimental.pallas import tpu_sc as plsc

_LANES = 16
_CHUNK = 128


def _build(B, S, V, D, NC, NS):
    N = B * S
    mesh = plsc.VectorSubcoreMesh(core_axis_name="c", subcore_axis_name="s")

    @functools.partial(
        pl.kernel,
        mesh=mesh,
        out_type=jax.ShapeDtypeStruct((N, D), jnp.float32),
        scratch_types=[
            pltpu.VMEM((B, _CHUNK), jnp.int32),
            pltpu.VMEM((_CHUNK, D), jnp.float32),
            pltpu.VMEM((B, _CHUNK, D), jnp.float32),
            pltpu.SemaphoreType.DMA,
            pltpu.SemaphoreType.DMA,
            pltpu.SemaphoreType.DMA,
            pltpu.SemaphoreType.DMA,
        ],
    )
    def emb(ids_hbm, tok_hbm, pos_hbm, out_hbm, idx_v, pos_v, tok_v, sem_p, sem_i, sem_g, sem_s):
        wid = lax.axis_index("s") * NC + lax.axis_index("c")
        off = wid * _CHUNK
        pos_cp = pltpu.async_copy(pos_hbm.at[pl.ds(off, _CHUNK)], pos_v, sem_p)
        idx_cps = [
            pltpu.async_copy(ids_hbm.at[b, pl.ds(off, _CHUNK)], idx_v.at[b], sem_i)
            for b in range(B)
        ]
        gathers = []
        for b in range(B):
            idx_cps[b].wait()
            gathers.append(
                pltpu.async_copy(tok_hbm.at[idx_v.at[b]], tok_v.at[b], sem_g)
            )
        pos_cp.wait()
        stores = []
        for b in range(B):
            gathers[b].wait()

            @plsc.parallel_loop(0, _CHUNK, step=1, unroll=4)
            def add_row(r):
                for c in range(D // _LANES):
                    v = pos_v[r, pl.ds(c * _LANES, _LANES)]
                    plsc.addupdate(tok_v.at[b, r, pl.ds(c * _LANES, _LANES)], v)

            stores.append(
                pltpu.async_copy(
                    tok_v.at[b], out_hbm.at[pl.ds(b * S + off, _CHUNK)], sem_s
                )
            )
        for b in range(B):
            stores[b].wait()

    return emb


def kernel(input_ids, token_weight, pos_weight):
    B, S = input_ids.shape
    V, D = token_weight.shape
    info = plsc.get_sparse_core_info()
    NC, NS = info.num_cores, info.num_subcores
    emb = _build(B, S, V, D, NC, NS)
    out = emb(input_ids.astype(jnp.int32), token_weight, pos_weight)
    return out.reshape(B, S, D)
